# baseline (speedup 1.0000x reference)
import functools

import jax
import jax.numpy as jnp
from jax import lax
from jax.experimental import pallas as pl
from jax.experimental.pallas import tpu as pltpu
from jax.experimental.pallas import tpu_sc as plsc

NC = 2
NS = 16
NW = NC * NS

EMB_D = 64
SPLITS = ((0, 104), (104, 96))
B_PER_IT = 4


def _gather_kernel(batch: int, seq: int):
    mesh = plsc.VectorSubcoreMesh(core_axis_name="c", subcore_axis_name="s",
                                  num_cores=NC, num_subcores=NS)
    b_per_w = batch // NW
    n_iters = b_per_w // B_PER_IT

    @functools.partial(
        pl.kernel,
        out_type=jax.ShapeDtypeStruct((batch, seq, EMB_D), jnp.float32),
        mesh=mesh,
        scratch_types=[
            pltpu.VMEM((b_per_w, seq), jnp.int32),
            pltpu.VMEM((B_PER_IT, seq, EMB_D), jnp.float32),
            pltpu.SemaphoreType.DMA,
        ],
        compiler_params=pltpu.CompilerParams(use_tc_tiling_on_sc=False),
    )
    def body(ids_hbm, table_hbm, out_hbm, idx_v, rows_v, gsem):
        wid = lax.axis_index("s") * NC + lax.axis_index("c")
        b_base = wid * b_per_w
        pltpu.sync_copy(ids_hbm.at[pl.ds(b_base, b_per_w)], idx_v)

        @pl.loop(0, n_iters)
        def _(it):
            copies = []
            for b in range(B_PER_IT):
                for off, ln in SPLITS:
                    copies.append(pltpu.async_copy(
                        table_hbm.at[idx_v.at[it * B_PER_IT + b, pl.ds(off, ln)]],
                        rows_v.at[b, pl.ds(off, ln)],
                        gsem,
                    ))
            for c in copies:
                c.wait()
            pltpu.sync_copy(
                rows_v,
                out_hbm.at[pl.ds(b_base + it * B_PER_IT, B_PER_IT)],
            )

    return body


def kernel(codec_ids, table):
    batch, seq = codec_ids.shape
    assert batch % (NW * B_PER_IT) == 0
    ids = codec_ids.astype(jnp.int32)
    return _gather_kernel(batch, seq)(ids, table)
